# baseline (speedup 1.0000x reference)
import jax
import jax.numpy as jnp
from jax.experimental import pallas as pl
from jax.experimental.pallas import tpu as pltpu

DIM_Z = 128
HID = 64
EDGE_DIM = 16
N = 512
BI = 128
BJ = 32
NBI = N // BI
NBJ = N // BJ
ROWS = BI * BJ
F32 = jnp.float32
HI = jax.lax.Precision.HIGHEST


def _body(z_ref, wihT_ref, whhT_ref, bih_ref, bhh_ref,
          egA_ref, egB_ref, b1_ref, m01_ref,
          a0x_ref, w0c_ref, a1x_ref, w1c_ref,
          a0w2_ref, a0b2_ref, a1w2_ref, a1b2_ref,
          c0w_ref, c0b_ref, c1w_ref, c1b_ref,
          out_ref,
          nodes_s, p_s, q_s, w0_s, m0_s, u1_s, out0_s, m1_s, w1_s):
    gi = jnp.dot(z_ref[:], wihT_ref[:], preferred_element_type=F32, precision=HI) + bih_ref[:]
    gir = gi[:, :HID]
    giz = gi[:, HID:2 * HID]
    gin = gi[:, 2 * HID:]

    def gru_step(t, h):
        gh = jnp.dot(h, whhT_ref[:], preferred_element_type=F32, precision=HI) + bhh_ref[:]
        r = jax.nn.sigmoid(gir + gh[:, :HID])
        zg = jax.nn.sigmoid(giz + gh[:, HID:2 * HID])
        n = jnp.tanh(gin + r * gh[:, 2 * HID:])
        h2 = (1.0 - zg) * n + zg * h
        nodes_s[pl.ds(t, 1), :] = h2
        return h2

    jax.lax.fori_loop(0, N, gru_step, jnp.zeros((1, HID), F32))

    nodes = nodes_s[:]
    p_s[:] = jnp.dot(nodes, egA_ref[:], preferred_element_type=F32, precision=HI)
    q_s[:] = jnp.dot(nodes, egB_ref[:], preferred_element_type=F32, precision=HI) + b1_ref[:]
    w0_s[:] = jnp.dot(nodes, a0x_ref[:], preferred_element_type=F32, precision=HI) + w0c_ref[:]
    m0_s[:] = jnp.dot(nodes, c0w_ref[:], preferred_element_type=F32, precision=HI) + c0b_ref[:]

    def phase_a_i(i, _):
        pt = p_s[pl.ds(i * BI, BI), :]
        w0t = w0_s[pl.ds(i * BI, BI), :][:, None, :]
        m0t = m0_s[pl.ds(i * BI, BI), :]
        ii = i * BI + jax.lax.broadcasted_iota(jnp.int32, (BI, BJ), 0)

        for J in range(NBJ):
            qt = q_s[J * BJ:(J + 1) * BJ, :]
            r = jnp.maximum(pt[:, None, :] + qt[None, :, :], 0.0)
            u = jnp.dot(r.reshape(ROWS, 2 * DIM_Z), m01_ref[:],
                        preferred_element_type=F32, precision=HI)
            u1_s[pl.ds(i * ROWS, ROWS), J * 8:(J + 1) * 8] = u[:, 8:16]
            pre = jnp.maximum(u[:, 0:8].reshape(BI, BJ, 8) + w0t, 0.0)
            alpha = (jnp.sum(pre * a0w2_ref[0][None, None, :], axis=2)
                     + a0b2_ref[0, 0])
            jj = J * BJ + jax.lax.broadcasted_iota(jnp.int32, (BI, BJ), 1)
            alpha = jnp.where(ii == jj, 0.0, alpha)
            acc = jax.lax.dot_general(alpha, m0t, (((0,), (0,)), ((), ())),
                                      preferred_element_type=F32, precision=HI)

            @pl.when(i == 0)
            def _():
                out0_s[J * BJ:(J + 1) * BJ, :] = acc

            @pl.when(i > 0)
            def _():
                out0_s[J * BJ:(J + 1) * BJ, :] += acc
        return 0

    jax.lax.fori_loop(0, NBI, phase_a_i, 0)

    n1 = jnp.maximum(out0_s[:], 0.0)
    m1_s[:] = jnp.dot(n1, c1w_ref[:], preferred_element_type=F32, precision=HI) + c1b_ref[:]
    w1_s[:] = jnp.dot(n1, a1x_ref[:], preferred_element_type=F32, precision=HI) + w1c_ref[:]

    w2l = jnp.concatenate([a1w2_ref[:]] * NBJ, axis=1)

    def phase_b_i(i, tot):
        w1t = w1_s[pl.ds(i * BI, BI), :]
        w1l = jnp.concatenate([w1t] * NBJ, axis=1)
        t3 = u1_s[pl.ds(i * ROWS, ROWS), :].reshape(BI, BJ, NBJ * 8)
        pre = jnp.maximum(t3 + w1l[:, None, :], 0.0)
        term = pre * w2l[:, None, :]
        total = jnp.sum(jnp.sum(term, axis=2), axis=1, keepdims=True)
        il = jax.lax.broadcasted_iota(jnp.int32, (BI, BJ, NBJ * 8), 0)
        jl = jax.lax.broadcasted_iota(jnp.int32, (BI, BJ, NBJ * 8), 1)
        ln = jax.lax.broadcasted_iota(jnp.int32, (BI, BJ, NBJ * 8), 2)
        diagmask = ((ln // 8) * BJ + jl) == (i * BI + il)
        diag = jnp.sum(jnp.sum(jnp.where(diagmask, term, 0.0), axis=2),
                       axis=1, keepdims=True)
        srow = total - diag + (N - 1) * a1b2_ref[0, 0]
        m1t = m1_s[pl.ds(i * BI, BI), :]
        return tot + jax.lax.dot_general(srow, m1t, (((0,), (0,)), ((), ())),
                                         preferred_element_type=F32, precision=HI)

    out_ref[:] = jax.lax.fori_loop(0, NBI, phase_b_i, jnp.zeros((1, HID), F32))


_SCRATCH = [
    pltpu.VMEM((N, HID), F32),
    pltpu.VMEM((N, 2 * DIM_Z), F32),
    pltpu.VMEM((N, 2 * DIM_Z), F32),
    pltpu.VMEM((N, 8), F32),
    pltpu.VMEM((N, HID), F32),
    pltpu.VMEM((NBI * ROWS, NBJ * 8), F32),
    pltpu.VMEM((N, HID), F32),
    pltpu.VMEM((N, HID), F32),
    pltpu.VMEM((N, 8), F32),
]

_OUT = jax.ShapeDtypeStruct((1, HID), F32)


def _prep(z, W_ih, W_hh, b_ih, b_hh, eg_w1, eg_b1, eg_w2, eg_b2,
          a0_w1, a0_b1, a0_w2, a0_b2, a1_w1, a1_b1, a1_w2, a1_b2,
          c0_w, c0_b, c1_w, c1_b):
    a01 = jnp.concatenate([a0_w1[:EDGE_DIM], a1_w1[:EDGE_DIM]], axis=1)
    m01 = eg_w2 @ a01
    c01 = eg_b2 @ a01
    w0c = (a0_b1 + c01[:8])[None]
    w1c = (a1_b1 + c01[8:])[None]
    return (z, W_ih.T, W_hh.T, b_ih[None], b_hh[None],
            eg_w1[:HID], eg_w1[HID:], eg_b1[None], m01,
            a0_w1[EDGE_DIM:], w0c, a1_w1[EDGE_DIM:], w1c,
            a0_w2.T, a0_b2[None], a1_w2.T, a1_b2[None],
            c0_w, c0_b[None], c1_w, c1_b[None])


@jax.jit
def kernel(z, W_ih, W_hh, b_ih, b_hh, eg_w1, eg_b1, eg_w2, eg_b2,
           a0_w1, a0_b1, a0_w2, a0_b2, a1_w1, a1_b1, a1_w2, a1_b2,
           c0_w, c0_b, c1_w, c1_b):
    args = _prep(z, W_ih, W_hh, b_ih, b_hh, eg_w1, eg_b1, eg_w2, eg_b2,
                 a0_w1, a0_b1, a0_w2, a0_b2, a1_w1, a1_b1, a1_w2, a1_b2,
                 c0_w, c0_b, c1_w, c1_b)
    out = pl.pallas_call(_body, out_shape=_OUT, scratch_shapes=_SCRATCH)(*args)
    return out[0]


# speedup vs baseline: 3.2259x; 3.2259x over previous
import jax
import jax.numpy as jnp
from jax.experimental import pallas as pl
from jax.experimental.pallas import tpu as pltpu

DIM_Z = 128
HID = 64
EDGE_DIM = 16
N = 512
BI = 128
BJ = 128
NBI = N // BI
NBJ = N // BJ
F32 = jnp.float32
HI = jax.lax.Precision.HIGHEST


def _body(z_ref, wihT_ref, whhT_ref, bih_ref, bhh_ref,
          egA_ref, egB_ref, b1_ref, m0p_ref, m1p_ref,
          a0x_ref, w0c_ref, a1x_ref, w1c_ref,
          a0w2_ref, a0b2_ref, a1w2_ref, a1b2_ref,
          c0w_ref, c0b_ref, c1w_ref, c1b_ref,
          out_ref,
          nodes_s, p_s, q_s, w0_s, m0_s, out0_s, m1_s, w1_s):
    gi = jnp.dot(z_ref[:], wihT_ref[:], preferred_element_type=F32, precision=HI) + bih_ref[:]
    gir = gi[:, :HID]
    giz = gi[:, HID:2 * HID]
    gin = gi[:, 2 * HID:]

    def gru_step(t, h):
        gh = jnp.dot(h, whhT_ref[:], preferred_element_type=F32, precision=HI) + bhh_ref[:]
        r = jax.nn.sigmoid(gir + gh[:, :HID])
        zg = jax.nn.sigmoid(giz + gh[:, HID:2 * HID])
        n = jnp.tanh(gin + r * gh[:, 2 * HID:])
        h2 = (1.0 - zg) * n + zg * h
        nodes_s[pl.ds(t, 1), :] = h2
        return h2

    jax.lax.fori_loop(0, N, gru_step, jnp.zeros((1, HID), F32))

    nodes = nodes_s[:]
    p_s[:] = jnp.dot(nodes, egA_ref[:], preferred_element_type=F32, precision=HI)
    q_s[:] = jnp.dot(nodes, egB_ref[:], preferred_element_type=F32, precision=HI) + b1_ref[:]
    w0_s[:] = jnp.dot(nodes, a0x_ref[:], preferred_element_type=F32, precision=HI) + w0c_ref[:]
    m0_s[:] = jnp.dot(nodes, c0w_ref[:], preferred_element_type=F32, precision=HI) + c0b_ref[:]

    def phase_a_i(i, _):
        pt = p_s[pl.ds(i * BI, BI), :]
        w0t = w0_s[pl.ds(i * BI, BI), :][:, None, :]
        m0t = m0_s[pl.ds(i * BI, BI), :]
        ii = i * BI + jax.lax.broadcasted_iota(jnp.int32, (BI, BJ), 0)

        def phase_a_j(j, _):
            qt = q_s[pl.ds(j * BJ, BJ), :]
            r = jnp.maximum(pt[:, None, :] + qt[None, :, :], 0.0)
            u = jnp.dot(r.reshape(BI * BJ, 2 * DIM_Z), m0p_ref[:],
                        preferred_element_type=F32)
            pre = jnp.maximum(u.reshape(BI, BJ, 8) + w0t, 0.0)
            alpha = (jnp.sum(pre * a0w2_ref[0][None, None, :], axis=2)
                     + a0b2_ref[0, 0])
            jj = j * BJ + jax.lax.broadcasted_iota(jnp.int32, (BI, BJ), 1)
            alpha = jnp.where(ii == jj, 0.0, alpha)
            acc = jax.lax.dot_general(alpha, m0t, (((0,), (0,)), ((), ())),
                                      preferred_element_type=F32)

            @pl.when(i == 0)
            def _():
                out0_s[pl.ds(j * BJ, BJ), :] = acc

            @pl.when(i > 0)
            def _():
                out0_s[pl.ds(j * BJ, BJ), :] += acc
            return 0

        jax.lax.fori_loop(0, NBJ, phase_a_j, 0)
        return 0

    jax.lax.fori_loop(0, NBI, phase_a_i, 0)

    n1 = jnp.maximum(out0_s[:], 0.0)
    m1_s[:] = jnp.dot(n1, c1w_ref[:], preferred_element_type=F32, precision=HI) + c1b_ref[:]
    w1_s[:] = jnp.dot(n1, a1x_ref[:], preferred_element_type=F32, precision=HI) + w1c_ref[:]

    def phase_b_i(i, tot):
        pt = p_s[pl.ds(i * BI, BI), :]
        w1t = w1_s[pl.ds(i * BI, BI), :][:, None, :]
        ii = i * BI + jax.lax.broadcasted_iota(jnp.int32, (BI, BJ), 0)

        def phase_b_j(j, srow):
            qt = q_s[pl.ds(j * BJ, BJ), :]
            r = jnp.maximum(pt[:, None, :] + qt[None, :, :], 0.0)
            u = jnp.dot(r.reshape(BI * BJ, 2 * DIM_Z), m1p_ref[:],
                        preferred_element_type=F32)
            pre = jnp.maximum(u.reshape(BI, BJ, 8) + w1t, 0.0)
            alpha = (jnp.sum(pre * a1w2_ref[0][None, None, :], axis=2)
                     + a1b2_ref[0, 0])
            jj = j * BJ + jax.lax.broadcasted_iota(jnp.int32, (BI, BJ), 1)
            alpha = jnp.where(ii == jj, 0.0, alpha)
            return srow + jnp.sum(alpha, axis=1, keepdims=True)

        srow = jax.lax.fori_loop(0, NBJ, phase_b_j, jnp.zeros((BI, 1), F32))
        m1t = m1_s[pl.ds(i * BI, BI), :]
        return tot + jax.lax.dot_general(srow, m1t, (((0,), (0,)), ((), ())),
                                         preferred_element_type=F32)

    out_ref[:] = jax.lax.fori_loop(0, NBI, phase_b_i, jnp.zeros((1, HID), F32))


_SCRATCH = [
    pltpu.VMEM((N, HID), F32),
    pltpu.VMEM((N, 2 * DIM_Z), F32),
    pltpu.VMEM((N, 2 * DIM_Z), F32),
    pltpu.VMEM((N, 8), F32),
    pltpu.VMEM((N, HID), F32),
    pltpu.VMEM((N, HID), F32),
    pltpu.VMEM((N, HID), F32),
    pltpu.VMEM((N, 8), F32),
]

_OUT = jax.ShapeDtypeStruct((1, HID), F32)


def _prep(z, W_ih, W_hh, b_ih, b_hh, eg_w1, eg_b1, eg_w2, eg_b2,
          a0_w1, a0_b1, a0_w2, a0_b2, a1_w1, a1_b1, a1_w2, a1_b2,
          c0_w, c0_b, c1_w, c1_b):
    m0p = eg_w2 @ a0_w1[:EDGE_DIM]
    m1p = eg_w2 @ a1_w1[:EDGE_DIM]
    w0c = (a0_b1 + eg_b2 @ a0_w1[:EDGE_DIM])[None]
    w1c = (a1_b1 + eg_b2 @ a1_w1[:EDGE_DIM])[None]
    return (z, W_ih.T, W_hh.T, b_ih[None], b_hh[None],
            eg_w1[:HID], eg_w1[HID:], eg_b1[None], m0p, m1p,
            a0_w1[EDGE_DIM:], w0c, a1_w1[EDGE_DIM:], w1c,
            a0_w2.T, a0_b2[None], a1_w2.T, a1_b2[None],
            c0_w, c0_b[None], c1_w, c1_b[None])


@jax.jit
def kernel(z, W_ih, W_hh, b_ih, b_hh, eg_w1, eg_b1, eg_w2, eg_b2,
           a0_w1, a0_b1, a0_w2, a0_b2, a1_w1, a1_b1, a1_w2, a1_b2,
           c0_w, c0_b, c1_w, c1_b):
    args = _prep(z, W_ih, W_hh, b_ih, b_hh, eg_w1, eg_b1, eg_w2, eg_b2,
                 a0_w1, a0_b1, a0_w2, a0_b2, a1_w1, a1_b1, a1_w2, a1_b2,
                 c0_w, c0_b, c1_w, c1_b)
    out = pl.pallas_call(_body, out_shape=_OUT, scratch_shapes=_SCRATCH)(*args)
    return out[0]
